# baseline (speedup 1.0000x reference)
import functools

import jax
import jax.numpy as jnp
from jax.experimental import pallas as pl

_BPB = 2
_NEG = -3.0e38


def _adj_kernel(s_ref, o_ref, *, n, other_k, bpb):
    r = bpb * n
    s = s_ref[...].reshape(r, n)

    col = jax.lax.broadcasted_iota(jnp.int32, (r, n), 1)
    row = jax.lax.broadcasted_iota(jnp.int32, (r, n), 0)
    diag = col == (row & (n - 1))

    w0 = jnp.where(diag, _NEG, s)
    m1 = jnp.max(w0, axis=-1, keepdims=True)
    work = w0
    mx = m1
    for _ in range(other_k - 1):
        work = jnp.where(work == mx, _NEG, work)
        mx = jnp.max(work, axis=-1, keepdims=True)

    e_all = jnp.exp(jnp.minimum(s - m1, 60.0))
    z = jnp.sum(e_all, axis=-1, keepdims=True)
    e_sel = jnp.where(w0 >= mx, e_all, 0.0)
    s7 = jnp.sum(e_sel, axis=-1, keepdims=True)
    sp = e_sel / (s7 + 1e-8 * z)

    eye = jnp.where(diag, 1.0, 0.0)
    for i in range(bpb):
        o_ref[i, :, 0, :] = eye[i * n : (i + 1) * n]
        o_ref[i, :, 1:, :] = jnp.broadcast_to(
            sp[i * n : (i + 1) * n, None, :], (n, other_k, n)
        )


def kernel(scores):
    b, n, _ = scores.shape
    total_k = 8
    bpb = _BPB if b % _BPB == 0 else 1
    grid = (b // bpb,)
    return pl.pallas_call(
        functools.partial(
            _adj_kernel, n=n, other_k=total_k - 1, bpb=bpb
        ),
        grid=grid,
        in_specs=[
            pl.BlockSpec((bpb, n, n), lambda bi: (bi, 0, 0)),
        ],
        out_specs=pl.BlockSpec(
            (bpb, n, total_k, n), lambda bi: (bi, 0, 0, 0)
        ),
        out_shape=jax.ShapeDtypeStruct((b, n, total_k, n), scores.dtype),
    )(scores)
